# baseline (speedup 1.0000x reference)
import jax
import jax.numpy as jnp
from jax import lax
from jax.experimental import pallas as pl
from jax.experimental.pallas import tpu as pltpu
from jax.experimental.pallas import tpu_sc as plsc

D_MODEL = 1024
D_FF = 512
N_EXP = 8
SEQ = 2048
SHARED_D_FF = 1024
TOP_K = 2

BT = 256
NBLK = SEQ * TOP_K // BT + N_EXP
PAD_ROWS = NBLK * BT
DH = D_MODEL // 2

NC = 2
NS = 16
NW = NC * NS
TPW = SEQ // NW
CPW = TPW // 16
GBT = 256


def _gate_shared_body(x_ref, wg_ref, ws1_ref, ws3_ref, ws2_ref,
                      i1_ref, i2_ref, w1_ref, w2_ref, o_ref):
    x = x_ref[...]
    logits = jnp.dot(x, wg_ref[...], preferred_element_type=jnp.float32)
    ids = lax.broadcasted_iota(jnp.int32, logits.shape, 1)
    a1 = jnp.argmax(logits, axis=1).astype(jnp.int32)
    l1 = jnp.max(logits, axis=1)
    masked = jnp.where(ids == a1[:, None], -1e30, logits)
    a2 = jnp.argmax(masked, axis=1).astype(jnp.int32)
    l2 = jnp.max(masked, axis=1)
    z = jnp.exp(l2 - l1)
    w1 = 1.0 / (1.0 + z)
    i1_ref[...] = a1
    i2_ref[...] = a2
    w1_ref[...] = w1
    w2_ref[...] = 1.0 - w1

    xb = x.astype(jnp.bfloat16)
    sh = jax.nn.silu(jnp.dot(xb, ws1_ref[...].astype(jnp.bfloat16),
                             preferred_element_type=jnp.float32))
    sh = sh * jnp.dot(xb, ws3_ref[...].astype(jnp.bfloat16),
                      preferred_element_type=jnp.float32)
    sh = jnp.dot(sh.astype(jnp.bfloat16), ws2_ref[...].astype(jnp.bfloat16),
                 preferred_element_type=jnp.float32)
    o_ref[...] = _pack_halves_tc(sh)


def _b16(s, dtype=jnp.int32):
    return lax.broadcast(s.astype(dtype) if hasattr(s, "astype") else
                         jnp.asarray(s, dtype), (16,))


def _pack_halves_tc(v):
    lo = lax.bitcast_convert_type(v[:, :DH].astype(jnp.bfloat16),
                                  jnp.int16).astype(jnp.int32) & 0xFFFF
    hi = lax.bitcast_convert_type(v[:, DH:].astype(jnp.bfloat16),
                                  jnp.int16).astype(jnp.int32) << 16
    return lo | hi


def _unpack_halves_tc(w):
    lo = lax.bitcast_convert_type(lax.shift_left(w, 16),
                                  jnp.float32).astype(jnp.bfloat16)
    hi = lax.bitcast_convert_type(w & jnp.int32(-65536),
                                  jnp.float32).astype(jnp.bfloat16)
    return lo, hi


def _dispatch_body(x_hbm, i1_hbm, i2_hbm, xs_hbm, p1_hbm, p2_hbm, blk_hbm,
                   i1_v, i2_v, xbuf, xb16, d1_v, d2_v, blk_v, sem_x, sem_s):
    wid = lax.axis_index("s") * NC + lax.axis_index("c")
    base = wid * TPW
    pltpu.sync_copy(i1_hbm, i1_v)
    pltpu.sync_copy(i2_hbm, i2_v)
    xcp = pltpu.async_copy(x_hbm.at[pl.ds(base, TPW)], xbuf, sem_x)

    lanes = lax.iota(jnp.int32, 16)
    my_first = wid * CPW

    def hist_step(i, carry):
        cnts, prefs = carry
        v1 = i1_v[pl.ds(i * 16, 16)]
        v2 = i2_v[pl.ds(i * 16, 16)]
        pred = _b16(i) < _b16(my_first)
        new_c = []
        new_p = []
        for e in range(N_EXP):
            ev = _b16(e)
            m = (v1 == ev).astype(jnp.int32) + (v2 == ev).astype(jnp.int32)
            new_c.append(cnts[e] + m)
            new_p.append(prefs[e] + jnp.where(pred, m,
                                              jnp.zeros((16,), jnp.int32)))
        return tuple(new_c), tuple(new_p)

    zero8 = tuple(jnp.zeros((16,), jnp.int32) for _ in range(N_EXP))
    cnts, prefs = lax.fori_loop(0, SEQ // 16, hist_step, (zero8, zero8))
    c = [_b16(jnp.sum(cnts[e])) for e in range(N_EXP)]
    p = [_b16(jnp.sum(prefs[e])) for e in range(N_EXP)]

    bt16 = jnp.full((16,), BT, jnp.int32)
    btm1 = jnp.full((16,), BT - 1, jnp.int32)
    sb = [jnp.zeros((16,), jnp.int32)] * N_EXP
    for e in range(1, N_EXP):
        sb[e] = sb[e - 1] + (c[e - 1] + btm1) // bt16

    run = [sb[e] * bt16 + p[e] for e in range(N_EXP)]

    for iv, dv in ((i1_v, d1_v), (i2_v, d2_v)):
        for cc in range(CPW):
            v = iv[pl.ds(base + cc * 16, 16)]
            dest = jnp.zeros((16,), jnp.int32)
            ones16 = jnp.ones((16,), jnp.int32)
            for e in range(N_EXP):
                m = v == _b16(e)
                mi = m.astype(jnp.int32)
                dest = jnp.where(m, run[e] + plsc.cumsum(mi) - ones16,
                                 dest)
                run[e] = run[e] + _b16(jnp.sum(mi))
            dv[pl.ds(cc * 16, 16)] = dest

    xcp.wait()

    rnd = jnp.full((16,), 0x7FFF, jnp.int32)
    one = jnp.ones((16,), jnp.int32)
    himask = jnp.full((16,), -65536, jnp.int32)

    def pack_grp(g, _):
        off = g * 16
        for r in range(TPW):
            ev = xbuf[r, pl.ds(off, 16)]
            ov = xbuf[r, pl.ds(DH + off, 16)]
            ei = plsc.bitcast(ev, jnp.int32)
            oi = plsc.bitcast(ov, jnp.int32)
            re = lax.shift_right_logical(
                ei + rnd + (lax.shift_right_logical(ei, 16) & one), 16)
            ro = (oi + rnd + (lax.shift_right_logical(oi, 16) & one)) & himask
            xb16[r, pl.ds(off, 16)] = re | ro
        return 0

    lax.fori_loop(0, DH // 16, pack_grp, 0)

    pltpu.async_copy(xb16, xs_hbm.at[d1_v], sem_s).wait()
    pltpu.async_copy(xb16, xs_hbm.at[d2_v], sem_s).wait()
    pltpu.sync_copy(d1_v, p1_hbm.at[pl.ds(base, TPW)])
    pltpu.sync_copy(d2_v, p2_hbm.at[pl.ds(base, TPW)])

    @pl.when(wid == 0)
    def _write_block_experts():
        nbu = sb[N_EXP - 1] + (c[N_EXP - 1] + btm1) // bt16
        for ch in range(NBLK // 16 + (1 if NBLK % 16 else 0)):
            bid = lanes + _b16(ch * 16)
            be = jnp.zeros((16,), jnp.int32)
            for e in range(1, N_EXP):
                be = be + (bid >= sb[e]).astype(jnp.int32)
            if ch == 1:
                be = jnp.where(lanes == 15, nbu, be)
            blk_v[pl.ds(ch * 16, 16)] = be
        pltpu.sync_copy(blk_v, blk_hbm)


def _grouped_body(be_ref, xs_ref, w1_ref, w3_ref, w2_ref, y_ref):
    @pl.when(pl.program_id(0) < be_ref[31])
    def _go():
        xlo, xhi = _unpack_halves_tc(xs_ref[...])
        w1 = w1_ref[0].astype(jnp.bfloat16)
        w3 = w3_ref[0].astype(jnp.bfloat16)
        h = jax.nn.silu(
            jnp.dot(xlo, w1[:DH], preferred_element_type=jnp.float32)
            + jnp.dot(xhi, w1[DH:], preferred_element_type=jnp.float32))
        h = h * (jnp.dot(xlo, w3[:DH], preferred_element_type=jnp.float32)
                 + jnp.dot(xhi, w3[DH:], preferred_element_type=jnp.float32))
        y = jnp.dot(h.astype(jnp.bfloat16), w2_ref[0].astype(jnp.bfloat16),
                    preferred_element_type=jnp.float32)
        y_ref[...] = _pack_halves_tc(y)


def _combine_body(y_hbm, p1_hbm, p2_hbm, w1_hbm, w2_hbm, sh_hbm, out_hbm,
                  p1_v, p2_v, w1_v, w2_v, y1_b, y2_b, sh_b, o_b,
                  sem1, sem2, sem3):
    wid = lax.axis_index("s") * NC + lax.axis_index("c")
    base = wid * TPW
    pltpu.sync_copy(p1_hbm.at[pl.ds(base, TPW)], p1_v)
    pltpu.sync_copy(p2_hbm.at[pl.ds(base, TPW)], p2_v)
    pltpu.sync_copy(w1_hbm.at[pl.ds(base, TPW)], w1_v)
    pltpu.sync_copy(w2_hbm.at[pl.ds(base, TPW)], w2_v)

    cps = []
    for cc in range(CPW):
        v1 = p1_v[pl.ds(cc * 16, 16)]
        v2 = p2_v[pl.ds(cc * 16, 16)]
        cp1 = pltpu.async_copy(y_hbm.at[v1], y1_b.at[cc], sem1)
        cp2 = pltpu.async_copy(y_hbm.at[v2], y2_b.at[cc], sem2)
        cp3 = pltpu.async_copy(sh_hbm.at[pl.ds(base + cc * 16, 16)],
                               sh_b.at[cc], sem3)
        cps.append((cp1, cp2, cp3))

    lanes = lax.iota(jnp.int32, 16)
    himask = jnp.full((16,), -65536, jnp.int32)
    zf = jnp.zeros((16,), jnp.float32)

    for cc in range(CPW):
        for cp in cps[cc]:
            cp.wait()
        w1c = w1_v[pl.ds(cc * 16, 16)]
        w2c = w2_v[pl.ds(cc * 16, 16)]
        for r in range(16):
            rv = _b16(r)
            wv1 = _b16(jnp.sum(jnp.where(lanes == rv, w1c, zf)), jnp.float32)
            wv2 = _b16(jnp.sum(jnp.where(lanes == rv, w2c, zf)), jnp.float32)

            def row_step(g, _, cc=cc, r=r, wv1=wv1, wv2=wv2):
                off = g * 16
                a1 = y1_b[cc, r, pl.ds(off, 16)]
                a2 = y2_b[cc, r, pl.ds(off, 16)]
                s = sh_b[cc, r, pl.ds(off, 16)]
                lo1 = plsc.bitcast(lax.shift_left(a1, 16), jnp.float32)
                hi1 = plsc.bitcast(a1 & himask, jnp.float32)
                lo2 = plsc.bitcast(lax.shift_left(a2, 16), jnp.float32)
                hi2 = plsc.bitcast(a2 & himask, jnp.float32)
                slo = plsc.bitcast(lax.shift_left(s, 16), jnp.float32)
                shi = plsc.bitcast(s & himask, jnp.float32)
                o_b[r, pl.ds(off, 16)] = wv1 * lo1 + wv2 * lo2 + slo
                o_b[r, pl.ds(DH + off, 16)] = wv1 * hi1 + wv2 * hi2 + shi
                return 0

            lax.fori_loop(0, DH // 16, row_step, 0)
        pltpu.sync_copy(o_b, out_hbm.at[pl.ds(base + cc * 16, 16)])


_sc_mesh = plsc.VectorSubcoreMesh(core_axis_name="c", subcore_axis_name="s",
                                  num_cores=NC, num_subcores=NS)

_dispatch = pl.kernel(
    _dispatch_body,
    out_type=(
        jax.ShapeDtypeStruct((PAD_ROWS, DH), jnp.int32),
        jax.ShapeDtypeStruct((SEQ,), jnp.int32),
        jax.ShapeDtypeStruct((SEQ,), jnp.int32),
        jax.ShapeDtypeStruct((32,), jnp.int32),
    ),
    mesh=_sc_mesh,
    compiler_params=pltpu.CompilerParams(needs_layout_passes=False),
    scratch_types=[
        pltpu.VMEM((SEQ,), jnp.int32),
        pltpu.VMEM((SEQ,), jnp.int32),
        pltpu.VMEM((TPW, D_MODEL), jnp.float32),
        pltpu.VMEM((TPW, DH), jnp.int32),
        pltpu.VMEM((TPW,), jnp.int32),
        pltpu.VMEM((TPW,), jnp.int32),
        pltpu.VMEM((32,), jnp.int32),
        pltpu.SemaphoreType.DMA,
        pltpu.SemaphoreType.DMA,
    ],
)

_combine = pl.kernel(
    _combine_body,
    out_type=jax.ShapeDtypeStruct((SEQ, D_MODEL), jnp.float32),
    mesh=_sc_mesh,
    compiler_params=pltpu.CompilerParams(needs_layout_passes=False),
    scratch_types=[
        pltpu.VMEM((TPW,), jnp.int32),
        pltpu.VMEM((TPW,), jnp.int32),
        pltpu.VMEM((TPW,), jnp.float32),
        pltpu.VMEM((TPW,), jnp.float32),
        pltpu.VMEM((CPW, 16, DH), jnp.int32),
        pltpu.VMEM((CPW, 16, DH), jnp.int32),
        pltpu.VMEM((CPW, 16, DH), jnp.int32),
        pltpu.VMEM((16, D_MODEL), jnp.float32),
        pltpu.SemaphoreType.DMA,
        pltpu.SemaphoreType.DMA,
        pltpu.SemaphoreType.DMA,
    ],
)


@jax.jit
def kernel(hidden_states, Wg, W1, W3, W2, Ws1, Ws3, Ws2):
    x = hidden_states.reshape(SEQ, D_MODEL)

    i1, i2, w1, w2, shared = pl.pallas_call(
        _gate_shared_body,
        grid=(SEQ // GBT,),
        in_specs=[
            pl.BlockSpec((GBT, D_MODEL), lambda i: (i, 0)),
            pl.BlockSpec((D_MODEL, N_EXP), lambda i: (0, 0)),
            pl.BlockSpec((D_MODEL, SHARED_D_FF), lambda i: (0, 0)),
            pl.BlockSpec((D_MODEL, SHARED_D_FF), lambda i: (0, 0)),
            pl.BlockSpec((SHARED_D_FF, D_MODEL), lambda i: (0, 0)),
        ],
        out_specs=[
            pl.BlockSpec((GBT,), lambda i: (i,)),
            pl.BlockSpec((GBT,), lambda i: (i,)),
            pl.BlockSpec((GBT,), lambda i: (i,)),
            pl.BlockSpec((GBT,), lambda i: (i,)),
            pl.BlockSpec((GBT, DH), lambda i: (i, 0)),
        ],
        out_shape=[
            jax.ShapeDtypeStruct((SEQ,), jnp.int32),
            jax.ShapeDtypeStruct((SEQ,), jnp.int32),
            jax.ShapeDtypeStruct((SEQ,), jnp.float32),
            jax.ShapeDtypeStruct((SEQ,), jnp.float32),
            jax.ShapeDtypeStruct((SEQ, DH), jnp.int32),
        ],
    )(x, Wg, Ws1, Ws3, Ws2)

    xs, pos1, pos2, blk = _dispatch(x, i1, i2)

    y = pl.pallas_call(
        _grouped_body,
        grid_spec=pltpu.PrefetchScalarGridSpec(
            num_scalar_prefetch=1,
            grid=(NBLK,),
            in_specs=[
                pl.BlockSpec((BT, DH),
                             lambda b, be: (jnp.minimum(b, be[31] - 1), 0)),
                pl.BlockSpec((1, D_MODEL, D_FF),
                             lambda b, be: (be[jnp.minimum(b, be[31] - 1)],
                                            0, 0)),
                pl.BlockSpec((1, D_MODEL, D_FF),
                             lambda b, be: (be[jnp.minimum(b, be[31] - 1)],
                                            0, 0)),
                pl.BlockSpec((1, D_FF, D_MODEL),
                             lambda b, be: (be[jnp.minimum(b, be[31] - 1)],
                                            0, 0)),
            ],
            out_specs=pl.BlockSpec((BT, DH),
                                   lambda b, be: (jnp.minimum(b, be[31] - 1),
                                                  0)),
        ),
        out_shape=jax.ShapeDtypeStruct((PAD_ROWS, DH), jnp.int32),
    )(blk, xs, W1, W3, W2)

    out = _combine(y, pos1, pos2, w1, w2, shared)
    return out.reshape(hidden_states.shape)


# speedup vs baseline: 1.0666x; 1.0666x over previous
import jax
import jax.numpy as jnp
from jax import lax
from jax.experimental import pallas as pl
from jax.experimental.pallas import tpu as pltpu
from jax.experimental.pallas import tpu_sc as plsc

D_MODEL = 1024
D_FF = 512
N_EXP = 8
SEQ = 2048
SHARED_D_FF = 1024
TOP_K = 2

BT = 256
NBLK = SEQ * TOP_K // BT + N_EXP
PAD_ROWS = NBLK * BT
DH = D_MODEL // 2

NC = 2
NS = 16
NW = NC * NS
TPW = SEQ // NW
CPW = TPW // 16
GBT = 256


def _gate_body(x_ref, wg_ref, i1_ref, i2_ref, w1_ref, w2_ref):
    x = x_ref[...]
    logits = jnp.dot(x, wg_ref[...], preferred_element_type=jnp.float32)
    ids = lax.broadcasted_iota(jnp.int32, logits.shape, 1)
    a1 = jnp.argmax(logits, axis=1).astype(jnp.int32)
    l1 = jnp.max(logits, axis=1)
    masked = jnp.where(ids == a1[:, None], -1e30, logits)
    a2 = jnp.argmax(masked, axis=1).astype(jnp.int32)
    l2 = jnp.max(masked, axis=1)
    z = jnp.exp(l2 - l1)
    w1 = 1.0 / (1.0 + z)
    i1_ref[...] = a1
    i2_ref[...] = a2
    w1_ref[...] = w1
    w2_ref[...] = 1.0 - w1


def _shared_body(x_ref, ws1_ref, ws3_ref, ws2_ref, o_ref):
    x = x_ref[...].astype(jnp.bfloat16)
    sh = jax.nn.silu(jnp.dot(x, ws1_ref[...].astype(jnp.bfloat16),
                             preferred_element_type=jnp.float32))
    sh = sh * jnp.dot(x, ws3_ref[...].astype(jnp.bfloat16),
                      preferred_element_type=jnp.float32)
    sh = jnp.dot(sh.astype(jnp.bfloat16), ws2_ref[...].astype(jnp.bfloat16),
                 preferred_element_type=jnp.float32)
    o_ref[...] = _pack_halves_tc(sh)


def _b16(s, dtype=jnp.int32):
    return lax.broadcast(s.astype(dtype) if hasattr(s, "astype") else
                         jnp.asarray(s, dtype), (16,))


def _pack_halves_tc(v):
    lo = lax.bitcast_convert_type(v[:, :DH].astype(jnp.bfloat16),
                                  jnp.int16).astype(jnp.int32) & 0xFFFF
    hi = lax.bitcast_convert_type(v[:, DH:].astype(jnp.bfloat16),
                                  jnp.int16).astype(jnp.int32) << 16
    return lo | hi


def _unpack_halves_tc(w):
    lo = lax.bitcast_convert_type(lax.shift_left(w, 16),
                                  jnp.float32).astype(jnp.bfloat16)
    hi = lax.bitcast_convert_type(w & jnp.int32(-65536),
                                  jnp.float32).astype(jnp.bfloat16)
    return lo, hi


def _dispatch_body(x_hbm, i1_hbm, i2_hbm, xs_hbm, p1_hbm, p2_hbm, blk_hbm,
                   i1_v, i2_v, xbuf, xb16, d1_v, d2_v, d1a, d2a, d1b, d2b,
                   blk_v, sem_x, sem_s, sem_p):
    wid = lax.axis_index("s") * NC + lax.axis_index("c")
    base = wid * TPW
    pltpu.sync_copy(i1_hbm, i1_v)
    pltpu.sync_copy(i2_hbm, i2_v)
    xcp = pltpu.async_copy(x_hbm.at[pl.ds(base, TPW)], xbuf, sem_x)

    lanes = lax.iota(jnp.int32, 16)
    my_first = wid * CPW

    def hist_step(i, carry):
        cnts, prefs = carry
        v1 = i1_v[pl.ds(i * 16, 16)]
        v2 = i2_v[pl.ds(i * 16, 16)]
        pred = _b16(i) < _b16(my_first)
        new_c = []
        new_p = []
        for e in range(N_EXP):
            ev = _b16(e)
            m = (v1 == ev).astype(jnp.int32) + (v2 == ev).astype(jnp.int32)
            new_c.append(cnts[e] + m)
            new_p.append(prefs[e] + jnp.where(pred, m,
                                              jnp.zeros((16,), jnp.int32)))
        return tuple(new_c), tuple(new_p)

    zero8 = tuple(jnp.zeros((16,), jnp.int32) for _ in range(N_EXP))
    cnts, prefs = lax.fori_loop(0, SEQ // 16, hist_step, (zero8, zero8))
    c = [_b16(jnp.sum(cnts[e])) for e in range(N_EXP)]
    p = [_b16(jnp.sum(prefs[e])) for e in range(N_EXP)]

    bt16 = jnp.full((16,), BT, jnp.int32)
    btm1 = jnp.full((16,), BT - 1, jnp.int32)
    sb = [jnp.zeros((16,), jnp.int32)] * N_EXP
    for e in range(1, N_EXP):
        sb[e] = sb[e - 1] + (c[e - 1] + btm1) // bt16

    run = [sb[e] * bt16 + p[e] for e in range(N_EXP)]

    for iv, dv, dva, dvb in ((i1_v, d1_v, d1a, d1b), (i2_v, d2_v, d2a, d2b)):
        for cc in range(CPW):
            v = iv[pl.ds(base + cc * 16, 16)]
            dest = jnp.zeros((16,), jnp.int32)
            ones16 = jnp.ones((16,), jnp.int32)
            for e in range(N_EXP):
                m = v == _b16(e)
                mi = m.astype(jnp.int32)
                dest = jnp.where(m, run[e] + plsc.cumsum(mi) - ones16,
                                 dest)
                run[e] = run[e] + _b16(jnp.sum(mi))
            dv[pl.ds(cc * 16, 16)] = dest
            half = dva if cc < CPW // 2 else dvb
            half[pl.ds((cc % (CPW // 2)) * 16, 16)] = dest

    pcp1 = pltpu.async_copy(d1_v, p1_hbm.at[pl.ds(base, TPW)], sem_p)
    pcp2 = pltpu.async_copy(d2_v, p2_hbm.at[pl.ds(base, TPW)], sem_p)
    xcp.wait()

    rnd = jnp.full((16,), 0x7FFF, jnp.int32)
    one = jnp.ones((16,), jnp.int32)
    himask = jnp.full((16,), -65536, jnp.int32)
    HR = TPW // 2

    def mk_pack(r0):
        def pack_grp(g, _):
            off = g * 16
            for r in range(r0, r0 + HR):
                ev = xbuf[r, pl.ds(off, 16)]
                ov = xbuf[r, pl.ds(DH + off, 16)]
                ei = plsc.bitcast(ev, jnp.int32)
                oi = plsc.bitcast(ov, jnp.int32)
                re = lax.shift_right_logical(
                    ei + rnd + (lax.shift_right_logical(ei, 16) & one), 16)
                ro = ((oi + rnd + (lax.shift_right_logical(oi, 16) & one))
                      & himask)
                xb16[r, pl.ds(off, 16)] = re | ro
            return 0
        return pack_grp

    lax.fori_loop(0, DH // 16, mk_pack(0), 0)
    sa1 = pltpu.async_copy(xb16.at[pl.ds(0, HR)], xs_hbm.at[d1a], sem_s)
    sa2 = pltpu.async_copy(xb16.at[pl.ds(0, HR)], xs_hbm.at[d2a], sem_s)
    lax.fori_loop(0, DH // 16, mk_pack(HR), 0)
    sb1 = pltpu.async_copy(xb16.at[pl.ds(HR, HR)], xs_hbm.at[d1b], sem_s)
    sb2 = pltpu.async_copy(xb16.at[pl.ds(HR, HR)], xs_hbm.at[d2b], sem_s)
    for cp in (sa1, sa2, sb1, sb2, pcp1, pcp2):
        cp.wait()

    @pl.when(wid == 0)
    def _write_block_experts():
        nbu = sb[N_EXP - 1] + (c[N_EXP - 1] + btm1) // bt16
        for ch in range(NBLK // 16 + (1 if NBLK % 16 else 0)):
            bid = lanes + _b16(ch * 16)
            be = jnp.zeros((16,), jnp.int32)
            for e in range(1, N_EXP):
                be = be + (bid >= sb[e]).astype(jnp.int32)
            if ch == 1:
                be = jnp.where(lanes == 15, nbu, be)
            blk_v[pl.ds(ch * 16, 16)] = be
        pltpu.sync_copy(blk_v, blk_hbm)


def _grouped_body(be_ref, xs_ref, w1_ref, w3_ref, w2_ref, y_ref):
    @pl.when(pl.program_id(0) < be_ref[31])
    def _go():
        xlo, xhi = _unpack_halves_tc(xs_ref[...])
        w1 = w1_ref[0].astype(jnp.bfloat16)
        w3 = w3_ref[0].astype(jnp.bfloat16)
        h = jax.nn.silu(
            jnp.dot(xlo, w1[:DH], preferred_element_type=jnp.float32)
            + jnp.dot(xhi, w1[DH:], preferred_element_type=jnp.float32))
        h = h * (jnp.dot(xlo, w3[:DH], preferred_element_type=jnp.float32)
                 + jnp.dot(xhi, w3[DH:], preferred_element_type=jnp.float32))
        y = jnp.dot(h.astype(jnp.bfloat16), w2_ref[0].astype(jnp.bfloat16),
                    preferred_element_type=jnp.float32)
        y_ref[...] = _pack_halves_tc(y)


def _combine_body(y_hbm, p1_hbm, p2_hbm, w1_hbm, w2_hbm, sh_hbm, out_hbm,
                  p1_v, p2_v, w1_v, w2_v, y1_b, y2_b, sh_b, o_b,
                  sem1, sem2, sem3):
    wid = lax.axis_index("s") * NC + lax.axis_index("c")
    base = wid * TPW
    pltpu.sync_copy(p1_hbm.at[pl.ds(base, TPW)], p1_v)
    pltpu.sync_copy(p2_hbm.at[pl.ds(base, TPW)], p2_v)
    pltpu.sync_copy(w1_hbm.at[pl.ds(base, TPW)], w1_v)
    pltpu.sync_copy(w2_hbm.at[pl.ds(base, TPW)], w2_v)

    cps = []
    for cc in range(CPW):
        v1 = p1_v[pl.ds(cc * 16, 16)]
        v2 = p2_v[pl.ds(cc * 16, 16)]
        cp1 = pltpu.async_copy(y_hbm.at[v1], y1_b.at[cc], sem1)
        cp2 = pltpu.async_copy(y_hbm.at[v2], y2_b.at[cc], sem2)
        cp3 = pltpu.async_copy(sh_hbm.at[pl.ds(base + cc * 16, 16)],
                               sh_b.at[cc], sem3)
        cps.append((cp1, cp2, cp3))

    lanes = lax.iota(jnp.int32, 16)
    himask = jnp.full((16,), -65536, jnp.int32)
    zf = jnp.zeros((16,), jnp.float32)

    for cc in range(CPW):
        for cp in cps[cc]:
            cp.wait()
        w1c = w1_v[pl.ds(cc * 16, 16)]
        w2c = w2_v[pl.ds(cc * 16, 16)]
        for r in range(16):
            rv = _b16(r)
            wv1 = _b16(jnp.sum(jnp.where(lanes == rv, w1c, zf)), jnp.float32)
            wv2 = _b16(jnp.sum(jnp.where(lanes == rv, w2c, zf)), jnp.float32)

            def row_step(g, _, cc=cc, r=r, wv1=wv1, wv2=wv2):
                off = g * 16
                a1 = y1_b[cc, r, pl.ds(off, 16)]
                a2 = y2_b[cc, r, pl.ds(off, 16)]
                s = sh_b[cc, r, pl.ds(off, 16)]
                lo1 = plsc.bitcast(lax.shift_left(a1, 16), jnp.float32)
                hi1 = plsc.bitcast(a1 & himask, jnp.float32)
                lo2 = plsc.bitcast(lax.shift_left(a2, 16), jnp.float32)
                hi2 = plsc.bitcast(a2 & himask, jnp.float32)
                slo = plsc.bitcast(lax.shift_left(s, 16), jnp.float32)
                shi = plsc.bitcast(s & himask, jnp.float32)
                o_b[r, pl.ds(off, 16)] = wv1 * lo1 + wv2 * lo2 + slo
                o_b[r, pl.ds(DH + off, 16)] = wv1 * hi1 + wv2 * hi2 + shi
                return 0

            lax.fori_loop(0, DH // 16, row_step, 0)
        pltpu.sync_copy(o_b, out_hbm.at[pl.ds(base + cc * 16, 16)])


_sc_mesh = plsc.VectorSubcoreMesh(core_axis_name="c", subcore_axis_name="s",
                                  num_cores=NC, num_subcores=NS)

_dispatch = pl.kernel(
    _dispatch_body,
    out_type=(
        jax.ShapeDtypeStruct((PAD_ROWS, DH), jnp.int32),
        jax.ShapeDtypeStruct((SEQ,), jnp.int32),
        jax.ShapeDtypeStruct((SEQ,), jnp.int32),
        jax.ShapeDtypeStruct((32,), jnp.int32),
    ),
    mesh=_sc_mesh,
    compiler_params=pltpu.CompilerParams(needs_layout_passes=False),
    scratch_types=[
        pltpu.VMEM((SEQ,), jnp.int32),
        pltpu.VMEM((SEQ,), jnp.int32),
        pltpu.VMEM((TPW, D_MODEL), jnp.float32),
        pltpu.VMEM((TPW, DH), jnp.int32),
        pltpu.VMEM((TPW,), jnp.int32),
        pltpu.VMEM((TPW,), jnp.int32),
        pltpu.VMEM((TPW // 2,), jnp.int32),
        pltpu.VMEM((TPW // 2,), jnp.int32),
        pltpu.VMEM((TPW // 2,), jnp.int32),
        pltpu.VMEM((TPW // 2,), jnp.int32),
        pltpu.VMEM((32,), jnp.int32),
        pltpu.SemaphoreType.DMA,
        pltpu.SemaphoreType.DMA,
        pltpu.SemaphoreType.DMA,
    ],
)

_combine = pl.kernel(
    _combine_body,
    out_type=jax.ShapeDtypeStruct((SEQ, D_MODEL), jnp.float32),
    mesh=_sc_mesh,
    compiler_params=pltpu.CompilerParams(needs_layout_passes=False),
    scratch_types=[
        pltpu.VMEM((TPW,), jnp.int32),
        pltpu.VMEM((TPW,), jnp.int32),
        pltpu.VMEM((TPW,), jnp.float32),
        pltpu.VMEM((TPW,), jnp.float32),
        pltpu.VMEM((CPW, 16, DH), jnp.int32),
        pltpu.VMEM((CPW, 16, DH), jnp.int32),
        pltpu.VMEM((CPW, 16, DH), jnp.int32),
        pltpu.VMEM((16, D_MODEL), jnp.float32),
        pltpu.SemaphoreType.DMA,
        pltpu.SemaphoreType.DMA,
        pltpu.SemaphoreType.DMA,
    ],
)


@jax.jit
def kernel(hidden_states, Wg, W1, W3, W2, Ws1, Ws3, Ws2):
    x = hidden_states.reshape(SEQ, D_MODEL)

    i1, i2, w1, w2 = pl.pallas_call(
        _gate_body,
        grid=(SEQ // GBT,),
        in_specs=[
            pl.BlockSpec((GBT, D_MODEL), lambda i: (i, 0)),
            pl.BlockSpec((D_MODEL, N_EXP), lambda i: (0, 0)),
        ],
        out_specs=[
            pl.BlockSpec((GBT,), lambda i: (i,)),
            pl.BlockSpec((GBT,), lambda i: (i,)),
            pl.BlockSpec((GBT,), lambda i: (i,)),
            pl.BlockSpec((GBT,), lambda i: (i,)),
        ],
        out_shape=[
            jax.ShapeDtypeStruct((SEQ,), jnp.int32),
            jax.ShapeDtypeStruct((SEQ,), jnp.int32),
            jax.ShapeDtypeStruct((SEQ,), jnp.float32),
            jax.ShapeDtypeStruct((SEQ,), jnp.float32),
        ],
    )(x, Wg)

    shared = pl.pallas_call(
        _shared_body,
        grid=(SEQ // GBT,),
        in_specs=[
            pl.BlockSpec((GBT, D_MODEL), lambda i: (i, 0)),
            pl.BlockSpec((D_MODEL, SHARED_D_FF), lambda i: (0, 0)),
            pl.BlockSpec((D_MODEL, SHARED_D_FF), lambda i: (0, 0)),
            pl.BlockSpec((SHARED_D_FF, D_MODEL), lambda i: (0, 0)),
        ],
        out_specs=pl.BlockSpec((GBT, DH), lambda i: (i, 0)),
        out_shape=jax.ShapeDtypeStruct((SEQ, DH), jnp.int32),
    )(x, Ws1, Ws3, Ws2)

    xs, pos1, pos2, blk = _dispatch(x, i1, i2)

    y = pl.pallas_call(
        _grouped_body,
        grid_spec=pltpu.PrefetchScalarGridSpec(
            num_scalar_prefetch=1,
            grid=(NBLK,),
            in_specs=[
                pl.BlockSpec((BT, DH),
                             lambda b, be: (jnp.minimum(b, be[31] - 1), 0)),
                pl.BlockSpec((1, D_MODEL, D_FF),
                             lambda b, be: (be[jnp.minimum(b, be[31] - 1)],
                                            0, 0)),
                pl.BlockSpec((1, D_MODEL, D_FF),
                             lambda b, be: (be[jnp.minimum(b, be[31] - 1)],
                                            0, 0)),
                pl.BlockSpec((1, D_FF, D_MODEL),
                             lambda b, be: (be[jnp.minimum(b, be[31] - 1)],
                                            0, 0)),
            ],
            out_specs=pl.BlockSpec((BT, DH),
                                   lambda b, be: (jnp.minimum(b, be[31] - 1),
                                                  0)),
        ),
        out_shape=jax.ShapeDtypeStruct((PAD_ROWS, DH), jnp.int32),
    )(blk, xs, W1, W3, W2)

    out = _combine(y, pos1, pos2, w1, w2, shared)
    return out.reshape(hidden_states.shape)


# speedup vs baseline: 1.0821x; 1.0145x over previous
import jax
import jax.numpy as jnp
from jax import lax
from jax.experimental import pallas as pl
from jax.experimental.pallas import tpu as pltpu
from jax.experimental.pallas import tpu_sc as plsc

D_MODEL = 1024
D_FF = 512
N_EXP = 8
SEQ = 2048
SHARED_D_FF = 1024
TOP_K = 2

BT = 128
NBLK = SEQ * TOP_K // BT + N_EXP
PAD_ROWS = NBLK * BT
DH = D_MODEL // 2

NC = 2
NS = 16
NW = NC * NS
TPW = SEQ // NW
CPW = TPW // 16
GBT = 256
NBLK_PAD = ((NBLK // 16) + 1) * 16
NBIDX = NBLK_PAD - 1


def _gate_body(x_ref, wg_ref, i1_ref, i2_ref, w1_ref, w2_ref):
    x = x_ref[...]
    logits = jnp.dot(x, wg_ref[...], preferred_element_type=jnp.float32)
    ids = lax.broadcasted_iota(jnp.int32, logits.shape, 1)
    a1 = jnp.argmax(logits, axis=1).astype(jnp.int32)
    l1 = jnp.max(logits, axis=1)
    masked = jnp.where(ids == a1[:, None], -1e30, logits)
    a2 = jnp.argmax(masked, axis=1).astype(jnp.int32)
    l2 = jnp.max(masked, axis=1)
    z = jnp.exp(l2 - l1)
    w1 = 1.0 / (1.0 + z)
    i1_ref[...] = a1
    i2_ref[...] = a2
    w1_ref[...] = w1
    w2_ref[...] = 1.0 - w1


def _shared_body(x_ref, ws1_ref, ws3_ref, ws2_ref, o_ref):
    x = x_ref[...].astype(jnp.bfloat16)
    sh = jax.nn.silu(jnp.dot(x, ws1_ref[...].astype(jnp.bfloat16),
                             preferred_element_type=jnp.float32))
    sh = sh * jnp.dot(x, ws3_ref[...].astype(jnp.bfloat16),
                      preferred_element_type=jnp.float32)
    sh = jnp.dot(sh.astype(jnp.bfloat16), ws2_ref[...].astype(jnp.bfloat16),
                 preferred_element_type=jnp.float32)
    o_ref[...] = _pack_halves_tc(sh)


def _b16(s, dtype=jnp.int32):
    return lax.broadcast(s.astype(dtype) if hasattr(s, "astype") else
                         jnp.asarray(s, dtype), (16,))


def _pack_halves_tc(v):
    lo = lax.bitcast_convert_type(v[:, :DH].astype(jnp.bfloat16),
                                  jnp.int16).astype(jnp.int32) & 0xFFFF
    hi = lax.bitcast_convert_type(v[:, DH:].astype(jnp.bfloat16),
                                  jnp.int16).astype(jnp.int32) << 16
    return lo | hi


def _unpack_halves_tc(w):
    lo = lax.bitcast_convert_type(lax.shift_left(w, 16),
                                  jnp.float32).astype(jnp.bfloat16)
    hi = lax.bitcast_convert_type(w & jnp.int32(-65536),
                                  jnp.float32).astype(jnp.bfloat16)
    return lo, hi


def _dispatch_body(x_hbm, i1_hbm, i2_hbm, xs_hbm, p1_hbm, p2_hbm, blk_hbm,
                   i1_v, i2_v, xbuf, xb16, d1_v, d2_v, d1a, d2a, d1b, d2b,
                   blk_v, sem_x, sem_s, sem_p):
    wid = lax.axis_index("s") * NC + lax.axis_index("c")
    base = wid * TPW
    pltpu.sync_copy(i1_hbm, i1_v)
    pltpu.sync_copy(i2_hbm, i2_v)
    xcp = pltpu.async_copy(x_hbm.at[pl.ds(base, TPW)], xbuf, sem_x)

    lanes = lax.iota(jnp.int32, 16)
    my_first = wid * CPW

    def hist_step(i, carry):
        cnts, prefs = carry
        v1 = i1_v[pl.ds(i * 16, 16)]
        v2 = i2_v[pl.ds(i * 16, 16)]
        pred = _b16(i) < _b16(my_first)
        new_c = []
        new_p = []
        for e in range(N_EXP):
            ev = _b16(e)
            m = (v1 == ev).astype(jnp.int32) + (v2 == ev).astype(jnp.int32)
            new_c.append(cnts[e] + m)
            new_p.append(prefs[e] + jnp.where(pred, m,
                                              jnp.zeros((16,), jnp.int32)))
        return tuple(new_c), tuple(new_p)

    zero8 = tuple(jnp.zeros((16,), jnp.int32) for _ in range(N_EXP))
    cnts, prefs = lax.fori_loop(0, SEQ // 16, hist_step, (zero8, zero8))
    c = [_b16(jnp.sum(cnts[e])) for e in range(N_EXP)]
    p = [_b16(jnp.sum(prefs[e])) for e in range(N_EXP)]

    bt16 = jnp.full((16,), BT, jnp.int32)
    btm1 = jnp.full((16,), BT - 1, jnp.int32)
    sb = [jnp.zeros((16,), jnp.int32)] * N_EXP
    for e in range(1, N_EXP):
        sb[e] = sb[e - 1] + (c[e - 1] + btm1) // bt16

    run = [sb[e] * bt16 + p[e] for e in range(N_EXP)]

    for iv, dv, dva, dvb in ((i1_v, d1_v, d1a, d1b), (i2_v, d2_v, d2a, d2b)):
        for cc in range(CPW):
            v = iv[pl.ds(base + cc * 16, 16)]
            dest = jnp.zeros((16,), jnp.int32)
            ones16 = jnp.ones((16,), jnp.int32)
            for e in range(N_EXP):
                m = v == _b16(e)
                mi = m.astype(jnp.int32)
                dest = jnp.where(m, run[e] + plsc.cumsum(mi) - ones16,
                                 dest)
                run[e] = run[e] + _b16(jnp.sum(mi))
            dv[pl.ds(cc * 16, 16)] = dest
            half = dva if cc < CPW // 2 else dvb
            half[pl.ds((cc % (CPW // 2)) * 16, 16)] = dest

    pcp1 = pltpu.async_copy(d1_v, p1_hbm.at[pl.ds(base, TPW)], sem_p)
    pcp2 = pltpu.async_copy(d2_v, p2_hbm.at[pl.ds(base, TPW)], sem_p)
    xcp.wait()

    rnd = jnp.full((16,), 0x7FFF, jnp.int32)
    one = jnp.ones((16,), jnp.int32)
    himask = jnp.full((16,), -65536, jnp.int32)
    HR = TPW // 2

    def mk_pack(r0):
        def pack_grp(g, _):
            off = g * 16
            for r in range(r0, r0 + HR):
                ev = xbuf[r, pl.ds(off, 16)]
                ov = xbuf[r, pl.ds(DH + off, 16)]
                ei = plsc.bitcast(ev, jnp.int32)
                oi = plsc.bitcast(ov, jnp.int32)
                re = lax.shift_right_logical(
                    ei + rnd + (lax.shift_right_logical(ei, 16) & one), 16)
                ro = ((oi + rnd + (lax.shift_right_logical(oi, 16) & one))
                      & himask)
                xb16[r, pl.ds(off, 16)] = re | ro
            return 0
        return pack_grp

    lax.fori_loop(0, DH // 16, mk_pack(0), 0)
    sa1 = pltpu.async_copy(xb16.at[pl.ds(0, HR)], xs_hbm.at[d1a], sem_s)
    sa2 = pltpu.async_copy(xb16.at[pl.ds(0, HR)], xs_hbm.at[d2a], sem_s)
    lax.fori_loop(0, DH // 16, mk_pack(HR), 0)
    sb1 = pltpu.async_copy(xb16.at[pl.ds(HR, HR)], xs_hbm.at[d1b], sem_s)
    sb2 = pltpu.async_copy(xb16.at[pl.ds(HR, HR)], xs_hbm.at[d2b], sem_s)
    for cp in (sa1, sa2, sb1, sb2, pcp1, pcp2):
        cp.wait()

    @pl.when(wid == 0)
    def _write_block_experts():
        nbu = sb[N_EXP - 1] + (c[N_EXP - 1] + btm1) // bt16
        nch = NBLK // 16 + (1 if NBLK % 16 else 0)
        for ch in range(nch):
            bid = lanes + _b16(ch * 16)
            be = jnp.zeros((16,), jnp.int32)
            for e in range(1, N_EXP):
                be = be + (bid >= sb[e]).astype(jnp.int32)
            if ch == nch - 1:
                be = jnp.where(lanes == 15, nbu, be)
            blk_v[pl.ds(ch * 16, 16)] = be
        pltpu.sync_copy(blk_v, blk_hbm)


def _grouped_body(be_ref, xs_ref, w1_ref, w3_ref, w2_ref, y_ref):
    @pl.when(pl.program_id(0) < be_ref[31])
    def _go():
        xlo, xhi = _unpack_halves_tc(xs_ref[...])
        w1 = w1_ref[0].astype(jnp.bfloat16)
        w3 = w3_ref[0].astype(jnp.bfloat16)
        h = jax.nn.silu(
            jnp.dot(xlo, w1[:DH], preferred_element_type=jnp.float32)
            + jnp.dot(xhi, w1[DH:], preferred_element_type=jnp.float32))
        h = h * (jnp.dot(xlo, w3[:DH], preferred_element_type=jnp.float32)
                 + jnp.dot(xhi, w3[DH:], preferred_element_type=jnp.float32))
        y = jnp.dot(h.astype(jnp.bfloat16), w2_ref[0].astype(jnp.bfloat16),
                    preferred_element_type=jnp.float32)
        y_ref[...] = _pack_halves_tc(y)


def _combine_body(y_hbm, p1_hbm, p2_hbm, w1_hbm, w2_hbm, sh_hbm, out_hbm,
                  p1_v, p2_v, w1_v, w2_v, y1_b, y2_b, sh_b, o_b,
                  sem1, sem2, sem3):
    wid = lax.axis_index("s") * NC + lax.axis_index("c")
    base = wid * TPW
    pltpu.sync_copy(p1_hbm.at[pl.ds(base, TPW)], p1_v)
    pltpu.sync_copy(p2_hbm.at[pl.ds(base, TPW)], p2_v)
    pltpu.sync_copy(w1_hbm.at[pl.ds(base, TPW)], w1_v)
    pltpu.sync_copy(w2_hbm.at[pl.ds(base, TPW)], w2_v)

    cps = []
    for cc in range(CPW):
        v1 = p1_v[pl.ds(cc * 16, 16)]
        v2 = p2_v[pl.ds(cc * 16, 16)]
        cp1 = pltpu.async_copy(y_hbm.at[v1], y1_b.at[cc], sem1)
        cp2 = pltpu.async_copy(y_hbm.at[v2], y2_b.at[cc], sem2)
        cp3 = pltpu.async_copy(sh_hbm.at[pl.ds(base + cc * 16, 16)],
                               sh_b.at[cc], sem3)
        cps.append((cp1, cp2, cp3))

    lanes = lax.iota(jnp.int32, 16)
    himask = jnp.full((16,), -65536, jnp.int32)
    zf = jnp.zeros((16,), jnp.float32)

    for cc in range(CPW):
        for cp in cps[cc]:
            cp.wait()
        w1c = w1_v[pl.ds(cc * 16, 16)]
        w2c = w2_v[pl.ds(cc * 16, 16)]
        for r in range(16):
            rv = _b16(r)
            wv1 = _b16(jnp.sum(jnp.where(lanes == rv, w1c, zf)), jnp.float32)
            wv2 = _b16(jnp.sum(jnp.where(lanes == rv, w2c, zf)), jnp.float32)

            def row_step(g, _, cc=cc, r=r, wv1=wv1, wv2=wv2):
                off = g * 16
                a1 = y1_b[cc, r, pl.ds(off, 16)]
                a2 = y2_b[cc, r, pl.ds(off, 16)]
                s = sh_b[cc, r, pl.ds(off, 16)]
                lo1 = plsc.bitcast(lax.shift_left(a1, 16), jnp.float32)
                hi1 = plsc.bitcast(a1 & himask, jnp.float32)
                lo2 = plsc.bitcast(lax.shift_left(a2, 16), jnp.float32)
                hi2 = plsc.bitcast(a2 & himask, jnp.float32)
                slo = plsc.bitcast(lax.shift_left(s, 16), jnp.float32)
                shi = plsc.bitcast(s & himask, jnp.float32)
                o_b[r, pl.ds(off, 16)] = wv1 * lo1 + wv2 * lo2 + slo
                o_b[r, pl.ds(DH + off, 16)] = wv1 * hi1 + wv2 * hi2 + shi
                return 0

            lax.fori_loop(0, DH // 16, row_step, 0)
        pltpu.sync_copy(o_b, out_hbm.at[pl.ds(base + cc * 16, 16)])


_sc_mesh = plsc.VectorSubcoreMesh(core_axis_name="c", subcore_axis_name="s",
                                  num_cores=NC, num_subcores=NS)

_dispatch = pl.kernel(
    _dispatch_body,
    out_type=(
        jax.ShapeDtypeStruct((PAD_ROWS, DH), jnp.int32),
        jax.ShapeDtypeStruct((SEQ,), jnp.int32),
        jax.ShapeDtypeStruct((SEQ,), jnp.int32),
        jax.ShapeDtypeStruct((NBLK_PAD,), jnp.int32),
    ),
    mesh=_sc_mesh,
    compiler_params=pltpu.CompilerParams(needs_layout_passes=False),
    scratch_types=[
        pltpu.VMEM((SEQ,), jnp.int32),
        pltpu.VMEM((SEQ,), jnp.int32),
        pltpu.VMEM((TPW, D_MODEL), jnp.float32),
        pltpu.VMEM((TPW, DH), jnp.int32),
        pltpu.VMEM((TPW,), jnp.int32),
        pltpu.VMEM((TPW,), jnp.int32),
        pltpu.VMEM((TPW // 2,), jnp.int32),
        pltpu.VMEM((TPW // 2,), jnp.int32),
        pltpu.VMEM((TPW // 2,), jnp.int32),
        pltpu.VMEM((TPW // 2,), jnp.int32),
        pltpu.VMEM((NBLK_PAD,), jnp.int32),
        pltpu.SemaphoreType.DMA,
        pltpu.SemaphoreType.DMA,
        pltpu.SemaphoreType.DMA,
    ],
)

_combine = pl.kernel(
    _combine_body,
    out_type=jax.ShapeDtypeStruct((SEQ, D_MODEL), jnp.float32),
    mesh=_sc_mesh,
    compiler_params=pltpu.CompilerParams(needs_layout_passes=False),
    scratch_types=[
        pltpu.VMEM((TPW,), jnp.int32),
        pltpu.VMEM((TPW,), jnp.int32),
        pltpu.VMEM((TPW,), jnp.float32),
        pltpu.VMEM((TPW,), jnp.float32),
        pltpu.VMEM((CPW, 16, DH), jnp.int32),
        pltpu.VMEM((CPW, 16, DH), jnp.int32),
        pltpu.VMEM((CPW, 16, DH), jnp.int32),
        pltpu.VMEM((16, D_MODEL), jnp.float32),
        pltpu.SemaphoreType.DMA,
        pltpu.SemaphoreType.DMA,
        pltpu.SemaphoreType.DMA,
    ],
)


@jax.jit
def kernel(hidden_states, Wg, W1, W3, W2, Ws1, Ws3, Ws2):
    x = hidden_states.reshape(SEQ, D_MODEL)

    i1, i2, w1, w2 = pl.pallas_call(
        _gate_body,
        grid=(SEQ // GBT,),
        in_specs=[
            pl.BlockSpec((GBT, D_MODEL), lambda i: (i, 0)),
            pl.BlockSpec((D_MODEL, N_EXP), lambda i: (0, 0)),
        ],
        out_specs=[
            pl.BlockSpec((GBT,), lambda i: (i,)),
            pl.BlockSpec((GBT,), lambda i: (i,)),
            pl.BlockSpec((GBT,), lambda i: (i,)),
            pl.BlockSpec((GBT,), lambda i: (i,)),
        ],
        out_shape=[
            jax.ShapeDtypeStruct((SEQ,), jnp.int32),
            jax.ShapeDtypeStruct((SEQ,), jnp.int32),
            jax.ShapeDtypeStruct((SEQ,), jnp.float32),
            jax.ShapeDtypeStruct((SEQ,), jnp.float32),
        ],
    )(x, Wg)

    shared = pl.pallas_call(
        _shared_body,
        grid=(SEQ // GBT,),
        in_specs=[
            pl.BlockSpec((GBT, D_MODEL), lambda i: (i, 0)),
            pl.BlockSpec((D_MODEL, SHARED_D_FF), lambda i: (0, 0)),
            pl.BlockSpec((D_MODEL, SHARED_D_FF), lambda i: (0, 0)),
            pl.BlockSpec((SHARED_D_FF, D_MODEL), lambda i: (0, 0)),
        ],
        out_specs=pl.BlockSpec((GBT, DH), lambda i: (i, 0)),
        out_shape=jax.ShapeDtypeStruct((SEQ, DH), jnp.int32),
    )(x, Ws1, Ws3, Ws2)

    xs, pos1, pos2, blk = _dispatch(x, i1, i2)

    y = pl.pallas_call(
        _grouped_body,
        grid_spec=pltpu.PrefetchScalarGridSpec(
            num_scalar_prefetch=1,
            grid=(NBLK,),
            in_specs=[
                pl.BlockSpec((BT, DH),
                             lambda b, be: (jnp.minimum(b, be[NBIDX] - 1), 0)),
                pl.BlockSpec((1, D_MODEL, D_FF),
                             lambda b, be: (be[jnp.minimum(b, be[NBIDX] - 1)],
                                            0, 0)),
                pl.BlockSpec((1, D_MODEL, D_FF),
                             lambda b, be: (be[jnp.minimum(b, be[NBIDX] - 1)],
                                            0, 0)),
                pl.BlockSpec((1, D_FF, D_MODEL),
                             lambda b, be: (be[jnp.minimum(b, be[NBIDX] - 1)],
                                            0, 0)),
            ],
            out_specs=pl.BlockSpec((BT, DH),
                                   lambda b, be: (jnp.minimum(b, be[NBIDX] - 1),
                                                  0)),
        ),
        out_shape=jax.ShapeDtypeStruct((PAD_ROWS, DH), jnp.int32),
    )(blk, xs, W1, W3, W2)

    out = _combine(y, pos1, pos2, w1, w2, shared)
    return out.reshape(hidden_states.shape)
